# baseline (speedup 1.0000x reference)
import functools

import jax
import jax.numpy as jnp
import numpy as np
from jax import lax
from jax.experimental import pallas as pl
from jax.experimental.pallas import tpu as pltpu
from jax.experimental.pallas import tpu_sc as plsc

NUM_CH = 256
BATCH = 16
ROW = 4096

PERM = np.array([
    121, 35, 130, 148, 197, 45, 176, 179, 139, 188, 99, 144, 152, 189, 31,
    112, 85, 63, 117, 174, 114, 254, 82, 65, 7, 4, 101, 102, 78, 163, 157,
    183, 29, 240, 177, 108, 83, 129, 212, 44, 211, 16, 58, 123, 37, 111, 19,
    61, 2, 142, 34, 156, 5, 90, 175, 167, 251, 110, 72, 155, 178, 219, 153,
    30, 42, 186, 246, 3, 70, 67, 223, 39, 56, 192, 169, 218, 195, 173, 245,
    241, 69, 80, 22, 6, 199, 118, 235, 54, 77, 147, 18, 249, 10, 11, 234, 53,
    236, 94, 32, 217, 159, 15, 184, 49, 137, 50, 138, 20, 237, 253, 185, 43,
    92, 8, 140, 233, 24, 81, 239, 96, 154, 135, 160, 106, 128, 191, 9, 200,
    40, 187, 71, 248, 164, 207, 93, 59, 201, 158, 210, 75, 131, 97, 66, 25,
    196, 242, 206, 243, 238, 73, 13, 52, 203, 202, 255, 194, 88, 250, 62,
    230, 150, 209, 132, 87, 76, 198, 60, 244, 47, 33, 79, 180, 247, 14, 228,
    17, 38, 86, 231, 190, 232, 23, 105, 220, 0, 145, 213, 226, 133, 41, 64,
    21, 161, 166, 124, 116, 26, 165, 168, 193, 57, 208, 181, 89, 146, 182,
    126, 125, 1, 115, 28, 113, 225, 172, 162, 48, 170, 227, 36, 252, 119,
    151, 120, 224, 122, 100, 91, 222, 55, 103, 51, 215, 127, 98, 107, 27, 74,
    136, 229, 204, 221, 12, 134, 109, 84, 205, 171, 143, 68, 216, 149, 141,
    104, 95, 214, 46,
], dtype=np.int32)

BT = 8
SC_B = BATCH - BT

NC = 2
NS = 16
NW = NC * NS
SC_ROWS = SC_B * NUM_CH
ROWS_PER_W = SC_ROWS // NW
CHUNK = 4
N_CHUNKS = ROWS_PER_W // CHUNK
NBUF = 7
AHEAD = 4

GIDX = (
    (np.arange(BT, BATCH, dtype=np.int32)[:, None] * NUM_CH + PERM[None, :])
    .reshape(-1, CHUNK)
)


def _sc_body(x_hbm, gidx_hbm, out_hbm, idx_v, bufs, gsems, ssems):
  wid = lax.axis_index("s") * NC + lax.axis_index("c")
  base = wid * ROWS_PER_W
  pltpu.sync_copy(gidx_hbm.at[pl.ds(wid * N_CHUNKS, N_CHUNKS)], idx_v)

  def gather(g, b):
    return pltpu.make_async_copy(
        x_hbm.at[idx_v.at[g]], bufs[b], gsems[b])

  def store(g, b):
    return pltpu.make_async_copy(
        bufs[b], out_hbm.at[pl.ds(base + g * CHUNK, CHUNK)], ssems[b])

  for g in range(AHEAD):
    gather(g, g % NBUF).start()
  for g in range(N_CHUNKS):
    b = g % NBUF
    gather(g, b).wait()
    store(g, b).start()
    ng = g + AHEAD
    if ng < N_CHUNKS:
      nb = ng % NBUF
      pg = ng - NBUF
      if pg >= 0:
        store(pg, nb).wait()
      gather(ng, nb).start()
  for g in range(N_CHUNKS - NBUF, N_CHUNKS):
    store(g, g % NBUF).wait()


def _sc_call(x_flat, gidx):
  call = pl.kernel(
      _sc_body,
      out_type=jax.ShapeDtypeStruct((SC_ROWS, ROW), jnp.float32),
      mesh=plsc.VectorSubcoreMesh(core_axis_name="c", subcore_axis_name="s"),
      scratch_types=[
          pltpu.VMEM((N_CHUNKS, CHUNK), jnp.int32),
          [pltpu.VMEM((CHUNK, ROW), jnp.float32) for _ in range(NBUF)],
          [pltpu.SemaphoreType.DMA for _ in range(NBUF)],
          [pltpu.SemaphoreType.DMA for _ in range(NBUF)],
      ],
  )
  return call(x_flat, gidx)


def _tc_body(perm_ref, x_ref, o_ref):
  o_ref[...] = x_ref[...]


def _tc_call(x_tc4):
  grid_spec = pltpu.PrefetchScalarGridSpec(
      num_scalar_prefetch=1,
      grid=(NUM_CH,),
      in_specs=[
          pl.BlockSpec((BT, 1, 32, 128), lambda c, perm: (0, perm[c], 0, 0)),
      ],
      out_specs=pl.BlockSpec((BT, 1, 32, 128), lambda c, perm: (0, c, 0, 0)),
  )
  return pl.pallas_call(
      _tc_body,
      grid_spec=grid_spec,
      out_shape=jax.ShapeDtypeStruct((BT, NUM_CH, 32, 128), jnp.float32),
  )(jnp.asarray(PERM), x_tc4)


@jax.jit
def kernel(x):
  x_flat = x.reshape(BATCH * NUM_CH, ROW)
  sc_out = _sc_call(x_flat, jnp.asarray(GIDX))
  tc_out = _tc_call(x[:BT].reshape(BT, NUM_CH, 32, 128))
  return jnp.concatenate(
      [tc_out.reshape(BT, NUM_CH, ROW), sc_out.reshape(SC_B, NUM_CH, ROW)],
      axis=0)


# speedup vs baseline: 6.5053x; 6.5053x over previous
import functools

import jax
import jax.numpy as jnp
import numpy as np
from jax import lax
from jax.experimental import pallas as pl
from jax.experimental.pallas import tpu as pltpu
from jax.experimental.pallas import tpu_sc as plsc

NUM_CH = 256
BATCH = 16
ROW = 4096

PERM = np.array([
    121, 35, 130, 148, 197, 45, 176, 179, 139, 188, 99, 144, 152, 189, 31,
    112, 85, 63, 117, 174, 114, 254, 82, 65, 7, 4, 101, 102, 78, 163, 157,
    183, 29, 240, 177, 108, 83, 129, 212, 44, 211, 16, 58, 123, 37, 111, 19,
    61, 2, 142, 34, 156, 5, 90, 175, 167, 251, 110, 72, 155, 178, 219, 153,
    30, 42, 186, 246, 3, 70, 67, 223, 39, 56, 192, 169, 218, 195, 173, 245,
    241, 69, 80, 22, 6, 199, 118, 235, 54, 77, 147, 18, 249, 10, 11, 234, 53,
    236, 94, 32, 217, 159, 15, 184, 49, 137, 50, 138, 20, 237, 253, 185, 43,
    92, 8, 140, 233, 24, 81, 239, 96, 154, 135, 160, 106, 128, 191, 9, 200,
    40, 187, 71, 248, 164, 207, 93, 59, 201, 158, 210, 75, 131, 97, 66, 25,
    196, 242, 206, 243, 238, 73, 13, 52, 203, 202, 255, 194, 88, 250, 62,
    230, 150, 209, 132, 87, 76, 198, 60, 244, 47, 33, 79, 180, 247, 14, 228,
    17, 38, 86, 231, 190, 232, 23, 105, 220, 0, 145, 213, 226, 133, 41, 64,
    21, 161, 166, 124, 116, 26, 165, 168, 193, 57, 208, 181, 89, 146, 182,
    126, 125, 1, 115, 28, 113, 225, 172, 162, 48, 170, 227, 36, 252, 119,
    151, 120, 224, 122, 100, 91, 222, 55, 103, 51, 215, 127, 98, 107, 27, 74,
    136, 229, 204, 221, 12, 134, 109, 84, 205, 171, 143, 68, 216, 149, 141,
    104, 95, 214, 46,
], dtype=np.int32)


def _tc_body(x_ref, o_ref):
  for c in range(NUM_CH):
    o_ref[0, c] = x_ref[0, int(PERM[c])]


def _tc_call(x):
  return pl.pallas_call(
      _tc_body,
      grid=(BATCH,),
      in_specs=[pl.BlockSpec((1, NUM_CH, ROW), lambda b: (b, 0, 0))],
      out_specs=pl.BlockSpec((1, NUM_CH, ROW), lambda b: (b, 0, 0)),
      out_shape=jax.ShapeDtypeStruct((BATCH, NUM_CH, ROW), jnp.float32),
  )(x)


@jax.jit
def kernel(x):
  return _tc_call(x)
